# initial kernel scaffold (unmeasured)
import functools

import jax
import jax.numpy as jnp
from jax import lax
from jax.experimental import pallas as pl
from jax.experimental.pallas import tpu as pltpu

N_DEV = 8

_printed_topo = False


def _debug_topo_once():
    global _printed_topo
    if _printed_topo:
        return
    _printed_topo = True
    import sys

    try:
        for d in jax.devices():
            sys.stderr.write(
                f"[kernel topo] id={d.id} coords={getattr(d, 'coords', None)} "
                f"core={getattr(d, 'core_on_chip', None)} kind={d.device_kind}\n"
            )
    except Exception as e:
        sys.stderr.write(f"[kernel topo] probe failed: {e!r}\n")


def kernel(x, w_mat):
    _debug_topo_once()
    kdim, mper = x.shape
    _, n = w_mat.shape
    assert kdim == N_DEV * mper

    xb = x.astype(jnp.bfloat16)
    wb = w_mat.astype(jnp.bfloat16)

    def body(x_ref, w_ref, out_ref, recv_ref, send_sems, recv_sems):
        my = lax.axis_index("i")

        barrier = pltpu.get_barrier_semaphore()
        for p in range(1, N_DEV):
            pl.semaphore_signal(
                barrier,
                inc=1,
                device_id=((my + p) % N_DEV,),
                device_id_type=pl.DeviceIdType.MESH,
            )
        pl.semaphore_wait(barrier, N_DEV - 1)

        sends = []
        for p in range(1, N_DEV):
            dst = (my + p) % N_DEV
            rdma = pltpu.make_async_remote_copy(
                src_ref=x_ref.at[pl.ds(dst * mper, mper), :],
                dst_ref=recv_ref.at[p - 1],
                send_sem=send_sems.at[p - 1],
                recv_sem=recv_sems.at[p - 1],
                device_id=(dst,),
                device_id_type=pl.DeviceIdType.MESH,
            )
            rdma.start()
            sends.append(rdma)

        out_ref[:, :] = jnp.dot(
            x_ref[pl.ds(my * mper, mper), :],
            w_ref[pl.ds(my * mper, mper), :],
            preferred_element_type=jnp.float32,
        )

        for p in range(1, N_DEV):
            src = (my - p) % N_DEV
            recv = pltpu.make_async_remote_copy(
                src_ref=x_ref.at[pl.ds(0, mper), :],
                dst_ref=recv_ref.at[p - 1],
                send_sem=send_sems.at[p - 1],
                recv_sem=recv_sems.at[p - 1],
                device_id=((my + p) % N_DEV,),
                device_id_type=pl.DeviceIdType.MESH,
            )
            recv.wait_recv()
            out_ref[:, :] += jnp.dot(
                recv_ref[p - 1],
                w_ref[pl.ds(src * mper, mper), :],
                preferred_element_type=jnp.float32,
            )

        for rdma in sends:
            rdma.wait_send()

        out_ref[:, :] = jax.nn.gelu(out_ref[:, :], approximate=True)

        @functools.partial(pl.run_scoped, exit_sem=pltpu.SemaphoreType.REGULAR)
        def _(exit_sem):
            for p in range(1, N_DEV):
                pl.semaphore_signal(
                    exit_sem,
                    inc=1,
                    device_id=((my + p) % N_DEV,),
                    device_id_type=pl.DeviceIdType.MESH,
                )
            pl.semaphore_wait(exit_sem, N_DEV - 1)

    return pl.pallas_call(
        body,
        out_shape=jax.ShapeDtypeStruct((mper, n), jnp.float32),
        in_specs=[
            pl.BlockSpec(memory_space=pltpu.VMEM),
            pl.BlockSpec(memory_space=pltpu.VMEM),
        ],
        out_specs=pl.BlockSpec(memory_space=pltpu.VMEM),
        scratch_shapes=[
            pltpu.VMEM((N_DEV - 1, mper, mper), jnp.bfloat16),
            pltpu.SemaphoreType.DMA((N_DEV - 1,)),
            pltpu.SemaphoreType.DMA((N_DEV - 1,)),
        ],
        compiler_params=pltpu.CompilerParams(
            collective_id=0,
            vmem_limit_bytes=128 * 1024 * 1024,
        ),
    )(xb, wb)


# baseline (device time: 149300 ns/iter reference)
import functools

import jax
import jax.numpy as jnp
from jax import lax
from jax.experimental import pallas as pl
from jax.experimental.pallas import tpu as pltpu

N_DEV = 8

_printed_topo = False


def _debug_topo_once():
    global _printed_topo
    if _printed_topo:
        return
    _printed_topo = True
    import sys

    try:
        for d in jax.devices():
            sys.stderr.write(
                f"[kernel topo] id={d.id} coords={getattr(d, 'coords', None)} "
                f"core={getattr(d, 'core_on_chip', None)} kind={d.device_kind}\n"
            )
    except Exception as e:
        sys.stderr.write(f"[kernel topo] probe failed: {e!r}\n")


def kernel(x, w_mat):
    _debug_topo_once()
    kdim, mper = x.shape
    _, n = w_mat.shape
    assert kdim == N_DEV * mper

    xb = x.astype(jnp.bfloat16)
    wb = w_mat.astype(jnp.bfloat16)

    def body(x_ref, w_ref, out_ref, recv_ref, wblk_ref, send_sems, recv_sems, wsems):
        my = lax.axis_index("i")

        def wcopy(j, slot):
            return pltpu.make_async_copy(
                w_ref.at[pl.ds(j * mper, mper), :],
                wblk_ref.at[slot],
                wsems.at[slot],
            )

        wcopy(my, 0).start()

        barrier = pltpu.get_barrier_semaphore()
        for p in range(1, N_DEV):
            pl.semaphore_signal(
                barrier,
                inc=1,
                device_id=((my + p) % N_DEV,),
                device_id_type=pl.DeviceIdType.MESH,
            )
        pl.semaphore_wait(barrier, N_DEV - 1)

        sends = []
        for p in range(1, N_DEV):
            dst = (my + p) % N_DEV
            rdma = pltpu.make_async_remote_copy(
                src_ref=x_ref.at[pl.ds(dst * mper, mper), :],
                dst_ref=recv_ref.at[p - 1],
                send_sem=send_sems.at[p - 1],
                recv_sem=recv_sems.at[p - 1],
                device_id=(dst,),
                device_id_type=pl.DeviceIdType.MESH,
            )
            rdma.start()
            sends.append(rdma)

        wcopy((my - 1) % N_DEV, 1).start()
        wcopy(my, 0).wait()
        out_ref[:, :] = jnp.dot(
            x_ref[pl.ds(my * mper, mper), :],
            wblk_ref[0],
            preferred_element_type=jnp.float32,
        )

        for p in range(1, N_DEV):
            slot = p % 2
            if p < N_DEV - 1:
                wcopy((my - p - 1) % N_DEV, (p + 1) % 2).start()
            recv = pltpu.make_async_remote_copy(
                src_ref=x_ref.at[pl.ds(0, mper), :],
                dst_ref=recv_ref.at[p - 1],
                send_sem=send_sems.at[p - 1],
                recv_sem=recv_sems.at[p - 1],
                device_id=((my + p) % N_DEV,),
                device_id_type=pl.DeviceIdType.MESH,
            )
            recv.wait_recv()
            wcopy((my - p) % N_DEV, slot).wait()
            out_ref[:, :] += jnp.dot(
                recv_ref[p - 1],
                wblk_ref[slot],
                preferred_element_type=jnp.float32,
            )

        for rdma in sends:
            rdma.wait_send()

        out_ref[:, :] = jax.nn.gelu(out_ref[:, :], approximate=True)

        @functools.partial(pl.run_scoped, exit_sem=pltpu.SemaphoreType.REGULAR)
        def _(exit_sem):
            for p in range(1, N_DEV):
                pl.semaphore_signal(
                    exit_sem,
                    inc=1,
                    device_id=((my + p) % N_DEV,),
                    device_id_type=pl.DeviceIdType.MESH,
                )
            pl.semaphore_wait(exit_sem, N_DEV - 1)

    return pl.pallas_call(
        body,
        out_shape=jax.ShapeDtypeStruct((mper, n), jnp.float32),
        in_specs=[
            pl.BlockSpec(memory_space=pltpu.VMEM),
            pl.BlockSpec(memory_space=pltpu.MemorySpace.HBM),
        ],
        out_specs=pl.BlockSpec(memory_space=pltpu.VMEM),
        scratch_shapes=[
            pltpu.VMEM((N_DEV - 1, mper, mper), jnp.bfloat16),
            pltpu.VMEM((2, mper, n), jnp.bfloat16),
            pltpu.SemaphoreType.DMA((N_DEV - 1,)),
            pltpu.SemaphoreType.DMA((N_DEV - 1,)),
            pltpu.SemaphoreType.DMA((2,)),
        ],
        compiler_params=pltpu.CompilerParams(
            collective_id=0,
            vmem_limit_bytes=128 * 1024 * 1024,
        ),
    )(xb, wb)


# device time: 85923 ns/iter; 1.7376x vs baseline; 1.7376x over previous
import functools

import jax

try:
    jax.config.update("jax_compilation_cache_dir", "/tmp/jax_persist_cache")
    jax.config.update("jax_persistent_cache_min_compile_time_secs", 1.0)
except Exception:
    pass

import jax.numpy as jnp
from jax import lax
from jax.experimental import pallas as pl
from jax.experimental.pallas import tpu as pltpu

N_DEV = 8
NCHUNK = 4096


def kernel(x, w_mat):
    kdim, mper = x.shape
    _, n = w_mat.shape
    assert kdim == N_DEV * mper
    n_chunks = n // NCHUNK

    def body(
        x_ref,
        w_ref,
        out_ref,
        xb_ref,
        recv_ref,
        wf32_ref,
        wb_ref,
        send_sems,
        recv_sems,
        wsems,
    ):
        my = lax.axis_index("i")

        def wcopy(j, c, slot):
            return pltpu.make_async_copy(
                w_ref.at[pl.ds(j * mper, mper), pl.ds(c * NCHUNK, NCHUNK)],
                wf32_ref.at[slot],
                wsems.at[slot],
            )

        def src_of(s):
            return (my - s) % N_DEV

        wcopy(src_of(0), 0, 0).start()

        xb_ref[:, :] = x_ref[:, :].astype(jnp.bfloat16)

        barrier = pltpu.get_barrier_semaphore()
        for p in range(1, N_DEV):
            pl.semaphore_signal(
                barrier,
                inc=1,
                device_id=((my + p) % N_DEV,),
                device_id_type=pl.DeviceIdType.MESH,
            )
        pl.semaphore_wait(barrier, N_DEV - 1)

        sends = []
        for p in range(1, N_DEV):
            dst = (my + p) % N_DEV
            rdma = pltpu.make_async_remote_copy(
                src_ref=xb_ref.at[pl.ds(dst * mper, mper), :],
                dst_ref=recv_ref.at[p - 1],
                send_sem=send_sems.at[p - 1],
                recv_sem=recv_sems.at[p - 1],
                device_id=(dst,),
                device_id_type=pl.DeviceIdType.MESH,
            )
            rdma.start()
            sends.append(rdma)

        n_steps = N_DEV * n_chunks

        def step_jc(t):
            return src_of(t // n_chunks), t % n_chunks

        for t in range(n_steps):
            s, c = t // n_chunks, t % n_chunks
            slot = t % 2
            if t + 1 < n_steps:
                j2, c2 = step_jc(t + 1)
                wcopy(j2, c2, (t + 1) % 2).start()
            if c == 0 and s > 0:
                recv = pltpu.make_async_remote_copy(
                    src_ref=xb_ref.at[pl.ds(0, mper), :],
                    dst_ref=recv_ref.at[s - 1],
                    send_sem=send_sems.at[s - 1],
                    recv_sem=recv_sems.at[s - 1],
                    device_id=((my + s) % N_DEV,),
                    device_id_type=pl.DeviceIdType.MESH,
                )
                recv.wait_recv()
            wcopy(0, 0, slot).wait()
            wb_ref[slot, :, :] = wf32_ref[slot, :, :].astype(jnp.bfloat16)
            if s == 0:
                a_blk = xb_ref[pl.ds(my * mper, mper), :]
            else:
                a_blk = recv_ref[s - 1]
            contrib = jnp.dot(
                a_blk, wb_ref[slot], preferred_element_type=jnp.float32
            )
            if s == 0:
                out_ref[:, pl.ds(c * NCHUNK, NCHUNK)] = contrib
            else:
                out_ref[:, pl.ds(c * NCHUNK, NCHUNK)] += contrib

        for rdma in sends:
            rdma.wait_send()

        out_ref[:, :] = jax.nn.gelu(out_ref[:, :], approximate=True)

        @functools.partial(pl.run_scoped, exit_sem=pltpu.SemaphoreType.REGULAR)
        def _(exit_sem):
            for p in range(1, N_DEV):
                pl.semaphore_signal(
                    exit_sem,
                    inc=1,
                    device_id=((my + p) % N_DEV,),
                    device_id_type=pl.DeviceIdType.MESH,
                )
            pl.semaphore_wait(exit_sem, N_DEV - 1)

    return pl.pallas_call(
        body,
        out_shape=jax.ShapeDtypeStruct((mper, n), jnp.float32),
        in_specs=[
            pl.BlockSpec(memory_space=pltpu.MemorySpace.VMEM),
            pl.BlockSpec(memory_space=pltpu.MemorySpace.HBM),
        ],
        out_specs=pl.BlockSpec(memory_space=pltpu.MemorySpace.VMEM),
        scratch_shapes=[
            pltpu.VMEM((kdim, mper), jnp.bfloat16),
            pltpu.VMEM((N_DEV - 1, mper, mper), jnp.bfloat16),
            pltpu.VMEM((2, mper, NCHUNK), jnp.float32),
            pltpu.VMEM((2, mper, NCHUNK), jnp.bfloat16),
            pltpu.SemaphoreType.DMA((N_DEV - 1,)),
            pltpu.SemaphoreType.DMA((N_DEV - 1,)),
            pltpu.SemaphoreType.DMA((2,)),
        ],
        compiler_params=pltpu.CompilerParams(
            collective_id=0,
            vmem_limit_bytes=100 * 1024 * 1024,
        ),
    )(x, w_mat)
